# baseline (speedup 1.0000x reference)
import functools

import jax
import jax.numpy as jnp
from jax import lax
from jax.experimental import pallas as pl
from jax.experimental.pallas import tpu as pltpu
from jax.experimental.pallas import tpu_sc as plsc

N = 10000
E = 160000
NPAD = 10240
FH = 128
NS = 16
T = E // NS
W = 40
NW = T // W
CHUNK = NPAD // NS
PW = 80
PNW = T // PW
ZR = 32
R = 1024
GRID = NPAD // R

_i32 = jnp.int32
_f32 = jnp.float32


def _mesh():
    return plsc.VectorSubcoreMesh(core_axis_name="c", subcore_axis_name="s")


def _prep_body(row_h, col_h, rowp_h, dinv_h, r2, c2, ones_b, dchunk,
               deg_acc, sem):
    c = lax.axis_index("c")
    s = lax.axis_index("s")
    lane = lax.iota(_i32, 16)

    @pl.when(c == 1)
    def _rowp():
        pltpu.sync_copy(row_h.at[s], r2)
        pltpu.sync_copy(col_h.at[s], c2)

        def win(w, _):
            def sub(i, _):
                r = r2[w, pl.ds(i * 16, 16)]
                cc = c2[w, pl.ds(i * 16, 16)]
                r2[w, pl.ds(i * 16, 16)] = jnp.where(r == cc, N + lane, r)
                return 0

            lax.fori_loop(0, PW // 16, sub, 0)
            return 0

        lax.fori_loop(0, PNW, win, 0)
        pltpu.sync_copy(r2, rowp_h.at[s])

    @pl.when(c == 0)
    def _deg():
        pltpu.sync_copy(row_h.at[s], r2)
        pltpu.sync_copy(col_h.at[s], c2)

        def fill(i, _):
            ones_b[pl.ds(i * 16, 16)] = jnp.full((16,), 1.0, _f32)
            return 0

        lax.fori_loop(0, PW // 16, fill, 0)

        def zero(i, _):
            dchunk[pl.ds(i * 16, 16)] = jnp.zeros((16,), _f32)
            return 0

        lax.fori_loop(0, CHUNK // 16, zero, 0)
        pltpu.sync_copy(dchunk, deg_acc.at[pl.ds(s * CHUNK, CHUNK)])

        def win(w, _):
            def sub(i, _):
                r = r2[w, pl.ds(i * 16, 16)]
                cc = c2[w, pl.ds(i * 16, 16)]
                c2[w, pl.ds(i * 16, 16)] = jnp.where(r == cc, N + lane, cc)
                return 0

            lax.fori_loop(0, PW // 16, sub, 0)
            return 0

        lax.fori_loop(0, PNW, win, 0)
        plsc.subcore_barrier()

        FK = 25

        def grp(g, _):
            def fire(j, _):
                pltpu.async_copy(ones_b, deg_acc.at[c2.at[g * FK + j]], sem,
                                 add=True)
                return 0

            lax.fori_loop(0, FK, fire, 0)

            def drain(j, _):
                pltpu.make_async_copy(
                    ones_b, deg_acc.at[c2.at[g * FK + j]], sem).wait()
                return 0

            lax.fori_loop(0, FK, drain, 0)
            return 0

        lax.fori_loop(0, PNW // FK, grp, 0)
        plsc.subcore_barrier()

        pltpu.sync_copy(deg_acc.at[pl.ds(s * CHUNK, CHUNK)], dchunk)

        def dv(i, _):
            d = dchunk[pl.ds(i * 16, 16)] + 1.0
            bits = lax.bitcast_convert_type(d, _i32)
            y = lax.bitcast_convert_type(
                jnp.int32(0x5F3759DF) - (bits >> 1), _f32)
            for _ in range(4):
                y = y * (1.5 - 0.5 * d * y * y)
            dchunk[pl.ds(i * 16, 16)] = y
            return 0

        lax.fori_loop(0, CHUNK // 16, dv, 0)
        pltpu.sync_copy(dchunk, dinv_h.at[pl.ds(s * CHUNK, CHUNK)])


def _prep(row3, col3):
    return pl.kernel(
        _prep_body,
        out_type=(
            jax.ShapeDtypeStruct((NS, PNW, PW), _i32),
            jax.ShapeDtypeStruct((NPAD,), _f32),
        ),
        mesh=_mesh(),
        scratch_types=[
            pltpu.VMEM((PNW, PW), _i32),
            pltpu.VMEM((PNW, PW), _i32),
            pltpu.VMEM((PW,), _f32),
            pltpu.VMEM((CHUNK,), _f32),
            pltpu.VMEM_SHARED((NPAD,), _f32),
            pltpu.SemaphoreType.DMA,
        ],
    )(row3, col3)


NB = 3


def _prop_body(tbl0, tbl1, rowp3, col2, out0, out1, idx2, cidx0, cidx1,
               cidx2b, gath0, gath1, gath2, acc,
               sg0, sg1, sg2, sc0, sc1, sc2):
    c = lax.axis_index("c")
    s = lax.axis_index("s")
    cidx = [cidx0, cidx1, cidx2b]
    gath = [gath0, gath1, gath2]
    sg = [sg0, sg1, sg2]
    sc = [sc0, sc1, sc2]

    def run(tbl, out):
        def z(i, _):
            def zl(j, _):
                gath0[i, pl.ds(j * 16, 16)] = jnp.zeros((16,), _f32)
                return 0

            lax.fori_loop(0, FH // 16, zl, 0)
            return 0

        lax.fori_loop(0, W, z, 0)

        def zc(k, _):
            pltpu.sync_copy(gath0, acc.at[pl.ds(s * CHUNK + k * W, W)])
            return 0

        lax.fori_loop(0, CHUNK // W, zc, 0)

        pltpu.sync_copy(rowp3.at[s], idx2)
        plsc.subcore_barrier()

        def g_start(w, j):
            pltpu.async_copy(tbl.at[idx2.at[w]], gath[j], sg[j])

        def g_wait(w, j):
            pltpu.make_async_copy(tbl.at[idx2.at[w]], gath[j], sg[j]).wait()

        def c_start(w, j):
            pltpu.async_copy(col2.at[s * NW + w], cidx[j], sc[j])

        def c_wait(w, j):
            pltpu.make_async_copy(col2.at[s * NW + w], cidx[j], sc[j]).wait()

        for j in range(NB):
            c_start(j, j)
            g_start(j, j)

        def win(k, _):
            for j in range(NB):
                w = NB * k + j
                g_wait(w, j)
                c_wait(w, j)
                pltpu.sync_copy(gath[j], acc.at[cidx[j]], add=True)

                @pl.when(w + NB < NW)
                def _():
                    c_start(w + NB, j)
                    g_start(w + NB, j)

            return 0

        lax.fori_loop(0, NW // NB, win, 0)
        for j in range(NW % NB):
            w = (NW // NB) * NB + j
            g_wait(w, j)
            c_wait(w, j)
            pltpu.sync_copy(gath[j], acc.at[cidx[j]], add=True)

        plsc.subcore_barrier()
        pltpu.sync_copy(acc.at[pl.ds(s * CHUNK, CHUNK)],
                        out.at[pl.ds(s * CHUNK, CHUNK)])

    @pl.when(c == 0)
    def _():
        run(tbl0, out0)

    @pl.when(c == 1)
    def _():
        run(tbl1, out1)


def _propagate(tbl0, tbl1, rowp3, col2):
    return pl.kernel(
        _prop_body,
        out_type=(
            jax.ShapeDtypeStruct((NPAD, FH), _f32),
            jax.ShapeDtypeStruct((NPAD, FH), _f32),
        ),
        mesh=_mesh(),
        scratch_types=(
            [pltpu.VMEM((NW, W), _i32)]
            + [pltpu.VMEM((W,), _i32) for _ in range(NB)]
            + [pltpu.VMEM((W, FH), _f32) for _ in range(NB)]
            + [pltpu.VMEM_SHARED((NPAD, FH), _f32)]
            + [pltpu.SemaphoreType.DMA for _ in range(2 * NB)]
        ),
    )(tbl0, tbl1, rowp3, col2)


def _scale_body(x_ref, dv_ref, xs0_ref, xs1_ref):
    i = pl.program_id(0)
    rowid = i * R + lax.broadcasted_iota(_i32, (R, 1), 0)
    xs = jnp.where(rowid < N, x_ref[...] * dv_ref[...], 0.0)
    xs0_ref[...] = xs[:, :FH]
    xs1_ref[...] = xs[:, FH:]


def _scale(x, dinv2):
    return pl.pallas_call(
        _scale_body,
        grid=(GRID,),
        in_specs=[
            pl.BlockSpec((R, 2 * FH), lambda i: (i, 0)),
            pl.BlockSpec((R, 1), lambda i: (i, 0)),
        ],
        out_specs=[
            pl.BlockSpec((R, FH), lambda i: (i, 0)),
            pl.BlockSpec((R, FH), lambda i: (i, 0)),
        ],
        out_shape=[
            jax.ShapeDtypeStruct((NPAD, FH), _f32),
            jax.ShapeDtypeStruct((NPAD, FH), _f32),
        ],
    )(x, dinv2)


def _mid_body(p0_ref, p1_ref, xs0_ref, xs1_ref, dv_ref, w1_ref, w2_ref,
              b1_ref, zs0_ref, zs1_ref):
    i = pl.program_id(0)
    dv = dv_ref[...]
    a = dv * (jnp.concatenate([p0_ref[...] + xs0_ref[...],
                               p1_ref[...] + xs1_ref[...]], axis=1))
    y = lax.dot_general(a, w1_ref[...], (((1,), (1,)), ((), ())),
                        preferred_element_type=_f32) + b1_ref[0]
    t = jnp.maximum(y, 0.0)
    w2 = w2_ref[...]
    wsum = w2[:, :512] + w2[:, 512:1024] + w2[:, 1024:]
    z = lax.dot_general(t, wsum, (((1,), (1,)), ((), ())),
                        preferred_element_type=_f32)
    rowid = i * R + lax.broadcasted_iota(_i32, (R, 1), 0)
    zs = jnp.where(rowid < N, z * dv, 0.0)
    zs0_ref[...] = zs[:, :FH]
    zs1_ref[...] = zs[:, FH:]


def _mid(p0, p1, xs0, xs1, dinv2, W1, W2, b1):
    return pl.pallas_call(
        _mid_body,
        grid=(GRID,),
        in_specs=[
            pl.BlockSpec((R, FH), lambda i: (i, 0)),
            pl.BlockSpec((R, FH), lambda i: (i, 0)),
            pl.BlockSpec((R, FH), lambda i: (i, 0)),
            pl.BlockSpec((R, FH), lambda i: (i, 0)),
            pl.BlockSpec((R, 1), lambda i: (i, 0)),
            pl.BlockSpec((512, 256), lambda i: (0, 0)),
            pl.BlockSpec((256, 1536), lambda i: (0, 0)),
            pl.BlockSpec((1, 1, 512), lambda i: (0, 0, 0)),
        ],
        out_specs=[
            pl.BlockSpec((R, FH), lambda i: (i, 0)),
            pl.BlockSpec((R, FH), lambda i: (i, 0)),
        ],
        out_shape=[
            jax.ShapeDtypeStruct((NPAD, FH), _f32),
            jax.ShapeDtypeStruct((NPAD, FH), _f32),
        ],
    )(p0, p1, xs0, xs1, dinv2, W1, W2, b1)


def _final_body(q0_ref, q1_ref, zs0_ref, zs1_ref, dv_ref, b2_ref, out_ref):
    y2 = dv_ref[...] * jnp.concatenate(
        [q0_ref[...] + zs0_ref[...], q1_ref[...] + zs1_ref[...]], axis=1
    ) + b2_ref[0]
    r = jnp.maximum(y2, 0.0)
    out_ref[...] = jnp.concatenate([r, r, r], axis=1)


def _final(q0, q1, zs0, zs1, dinv2, b2):
    return pl.pallas_call(
        _final_body,
        grid=(GRID,),
        in_specs=[
            pl.BlockSpec((R, FH), lambda i: (i, 0)),
            pl.BlockSpec((R, FH), lambda i: (i, 0)),
            pl.BlockSpec((R, FH), lambda i: (i, 0)),
            pl.BlockSpec((R, FH), lambda i: (i, 0)),
            pl.BlockSpec((R, 1), lambda i: (i, 0)),
            pl.BlockSpec((1, 1, 256), lambda i: (0, 0, 0)),
        ],
        out_specs=pl.BlockSpec((R, 768), lambda i: (i, 0)),
        out_shape=jax.ShapeDtypeStruct((N, 768), _f32),
    )(q0, q1, zs0, zs1, dinv2, b2)


def kernel(x, edge_index, split, W1, W2, bias1, bias2):
    del split
    assert x.shape == (N, 256) and edge_index.shape == (2, E)
    row3 = edge_index[0].astype(_i32).reshape(NS, PNW, PW)
    col3 = edge_index[1].astype(_i32).reshape(NS, PNW, PW)
    rowp3p, dinv = _prep(row3, col3)
    rowp3 = rowp3p.reshape(NS, NW, W)
    col2 = col3.reshape(NS * NW, W)
    dinv2 = dinv.reshape(NPAD, 1)
    xs0, xs1 = _scale(x, dinv2)
    p0, p1 = _propagate(xs0, xs1, rowp3, col2)
    zs0, zs1 = _mid(p0, p1, xs0, xs1, dinv2, W1, W2,
                    bias1.reshape(1, 1, 512))
    q0, q1 = _propagate(zs0, zs1, rowp3, col2)
    return _final(q0, q1, zs0, zs1, dinv2, bias2.reshape(1, 1, 256))


# speedup vs baseline: 1.0220x; 1.0220x over previous
import functools

import jax
import jax.numpy as jnp
from jax import lax
from jax.experimental import pallas as pl
from jax.experimental.pallas import tpu as pltpu
from jax.experimental.pallas import tpu_sc as plsc

N = 10000
E = 160000
NPAD = 10240
FH = 128
NS = 16
T = E // NS
W = 80
NW = T // W
CHUNK = NPAD // NS
ZR = 32
R = 1024
GRID = NPAD // R

_i32 = jnp.int32
_f32 = jnp.float32


def _mesh():
    return plsc.VectorSubcoreMesh(core_axis_name="c", subcore_axis_name="s")


def _prep_body(edge_h, rowp_h, colp_h, dinv_h, r2, c2, ones_b, dchunk,
               deg_acc, sem):
    c = lax.axis_index("c")
    s = lax.axis_index("s")
    lane = lax.iota(_i32, 16)

    @pl.when(c == 1)
    def _rowp():
        pltpu.sync_copy(edge_h.at[0, s], r2)
        pltpu.sync_copy(edge_h.at[1, s], c2)

        def win(w, _):
            def sub(i, _):
                r = r2[w, pl.ds(i * 16, 16)]
                cc = c2[w, pl.ds(i * 16, 16)]
                r2[w, pl.ds(i * 16, 16)] = jnp.where(r == cc, N + lane, r)
                return 0

            lax.fori_loop(0, W // 16, sub, 0)
            return 0

        lax.fori_loop(0, NW, win, 0)
        pltpu.sync_copy(r2, rowp_h.at[s])

    @pl.when(c == 0)
    def _deg():
        pltpu.sync_copy(edge_h.at[0, s], r2)
        pltpu.sync_copy(edge_h.at[1, s], c2)

        def fill(i, _):
            ones_b[pl.ds(i * 16, 16)] = jnp.full((16,), 1.0, _f32)
            return 0

        lax.fori_loop(0, W // 16, fill, 0)

        def zero(i, _):
            dchunk[pl.ds(i * 16, 16)] = jnp.zeros((16,), _f32)
            return 0

        lax.fori_loop(0, CHUNK // 16, zero, 0)
        pltpu.sync_copy(dchunk, deg_acc.at[pl.ds(s * CHUNK, CHUNK)])

        def win(w, _):
            def sub(i, _):
                r = r2[w, pl.ds(i * 16, 16)]
                cc = c2[w, pl.ds(i * 16, 16)]
                c2[w, pl.ds(i * 16, 16)] = jnp.where(r == cc, N + lane, cc)
                return 0

            lax.fori_loop(0, W // 16, sub, 0)
            return 0

        lax.fori_loop(0, NW, win, 0)
        plsc.subcore_barrier()

        FK = 25

        def grp(g, _):
            def fire(j, _):
                pltpu.async_copy(ones_b, deg_acc.at[c2.at[g * FK + j]], sem,
                                 add=True)
                return 0

            lax.fori_loop(0, FK, fire, 0)

            def drain(j, _):
                pltpu.make_async_copy(
                    ones_b, deg_acc.at[c2.at[g * FK + j]], sem).wait()
                return 0

            lax.fori_loop(0, FK, drain, 0)
            return 0

        lax.fori_loop(0, NW // FK, grp, 0)
        pltpu.sync_copy(c2, colp_h.at[s])
        plsc.subcore_barrier()

        pltpu.sync_copy(deg_acc.at[pl.ds(s * CHUNK, CHUNK)], dchunk)

        def dv(i, _):
            d = dchunk[pl.ds(i * 16, 16)] + 1.0
            bits = lax.bitcast_convert_type(d, _i32)
            y = lax.bitcast_convert_type(
                jnp.int32(0x5F3759DF) - (bits >> 1), _f32)
            for _ in range(4):
                y = y * (1.5 - 0.5 * d * y * y)
            dchunk[pl.ds(i * 16, 16)] = y
            return 0

        lax.fori_loop(0, CHUNK // 16, dv, 0)
        pltpu.sync_copy(dchunk, dinv_h.at[pl.ds(s * CHUNK, CHUNK)])


def _prep(edge4):
    return pl.kernel(
        _prep_body,
        out_type=(
            jax.ShapeDtypeStruct((NS, NW, W), _i32),
            jax.ShapeDtypeStruct((NS, NW, W), _i32),
            jax.ShapeDtypeStruct((NPAD,), _f32),
        ),
        mesh=_mesh(),
        scratch_types=[
            pltpu.VMEM((NW, W), _i32),
            pltpu.VMEM((NW, W), _i32),
            pltpu.VMEM((W,), _f32),
            pltpu.VMEM((CHUNK,), _f32),
            pltpu.VMEM_SHARED((NPAD,), _f32),
            pltpu.SemaphoreType.DMA,
        ],
    )(edge4)


def _prop_body(tbl0, tbl1, rowp3, col2, out0, out1, idx2, cidxa, cidxb,
               gatha, gathb, acc, sga, sgb, sca, scb):
    c = lax.axis_index("c")
    s = lax.axis_index("s")

    def run(tbl, out):
        def z(i, _):
            def zl(j, _):
                gatha[i, pl.ds(j * 16, 16)] = jnp.zeros((16,), _f32)
                return 0

            lax.fori_loop(0, FH // 16, zl, 0)
            return 0

        lax.fori_loop(0, W, z, 0)

        def zc(k, _):
            pltpu.sync_copy(gatha, acc.at[pl.ds(s * CHUNK + k * W, W)])
            return 0

        lax.fori_loop(0, CHUNK // W, zc, 0)

        pltpu.sync_copy(rowp3.at[s], idx2)
        plsc.subcore_barrier()

        def g_start(w, buf, sem):
            pltpu.async_copy(tbl.at[idx2.at[w]], buf, sem)

        def g_wait(w, buf, sem):
            pltpu.make_async_copy(tbl.at[idx2.at[w]], buf, sem).wait()

        def c_start(w, buf, sem):
            pltpu.async_copy(col2.at[s * NW + w], buf, sem)

        def c_wait(w, buf, sem):
            pltpu.make_async_copy(col2.at[s * NW + w], buf, sem).wait()

        c_start(0, cidxa, sca)
        c_start(1, cidxb, scb)
        g_start(0, gatha, sga)
        g_start(1, gathb, sgb)

        def win(k, _):
            a = 2 * k
            b = 2 * k + 1
            g_wait(a, gatha, sga)
            c_wait(a, cidxa, sca)
            pltpu.sync_copy(gatha, acc.at[cidxa], add=True)
            c_start(a + 2, cidxa, sca)
            g_start(a + 2, gatha, sga)
            g_wait(b, gathb, sgb)
            c_wait(b, cidxb, scb)
            pltpu.sync_copy(gathb, acc.at[cidxb], add=True)

            @pl.when(b + 2 < NW)
            def _():
                c_start(b + 2, cidxb, scb)
                g_start(b + 2, gathb, sgb)

            return 0

        lax.fori_loop(0, NW // 2, win, 0)
        g_wait(NW - 1, gatha, sga)
        c_wait(NW - 1, cidxa, sca)
        pltpu.sync_copy(gatha, acc.at[cidxa], add=True)

        plsc.subcore_barrier()
        pltpu.sync_copy(acc.at[pl.ds(s * CHUNK, CHUNK)],
                        out.at[pl.ds(s * CHUNK, CHUNK)])

    @pl.when(c == 0)
    def _():
        run(tbl0, out0)

    @pl.when(c == 1)
    def _():
        run(tbl1, out1)


def _propagate(tbl0, tbl1, rowp3, col2):
    return pl.kernel(
        _prop_body,
        out_type=(
            jax.ShapeDtypeStruct((NPAD, FH), _f32),
            jax.ShapeDtypeStruct((NPAD, FH), _f32),
        ),
        mesh=_mesh(),
        scratch_types=[
            pltpu.VMEM((NW, W), _i32),
            pltpu.VMEM((W,), _i32),
            pltpu.VMEM((W,), _i32),
            pltpu.VMEM((W, FH), _f32),
            pltpu.VMEM((W, FH), _f32),
            pltpu.VMEM_SHARED((NPAD, FH), _f32),
            pltpu.SemaphoreType.DMA,
            pltpu.SemaphoreType.DMA,
            pltpu.SemaphoreType.DMA,
            pltpu.SemaphoreType.DMA,
        ],
    )(tbl0, tbl1, rowp3, col2)


def _scale_body(x_ref, dv_ref, xs0_ref, xs1_ref):
    i = pl.program_id(0)
    rowid = i * R + lax.broadcasted_iota(_i32, (R, 1), 0)
    xs = jnp.where(rowid < N, x_ref[...] * dv_ref[...], 0.0)
    xs0_ref[...] = xs[:, :FH]
    xs1_ref[...] = xs[:, FH:]


def _scale(x, dinv2):
    return pl.pallas_call(
        _scale_body,
        grid=(GRID,),
        in_specs=[
            pl.BlockSpec((R, 2 * FH), lambda i: (i, 0)),
            pl.BlockSpec((R, 1), lambda i: (i, 0)),
        ],
        out_specs=[
            pl.BlockSpec((R, FH), lambda i: (i, 0)),
            pl.BlockSpec((R, FH), lambda i: (i, 0)),
        ],
        out_shape=[
            jax.ShapeDtypeStruct((NPAD, FH), _f32),
            jax.ShapeDtypeStruct((NPAD, FH), _f32),
        ],
    )(x, dinv2)


def _mid_body(p0_ref, p1_ref, xs0_ref, xs1_ref, dv_ref, w1_ref, w2_ref,
              b1_ref, zs0_ref, zs1_ref):
    i = pl.program_id(0)
    dv = dv_ref[...]
    a = dv * (jnp.concatenate([p0_ref[...] + xs0_ref[...],
                               p1_ref[...] + xs1_ref[...]], axis=1))
    y = lax.dot_general(a, w1_ref[...], (((1,), (1,)), ((), ())),
                        preferred_element_type=_f32) + b1_ref[0]
    t = jnp.maximum(y, 0.0)
    w2 = w2_ref[...]
    wsum = w2[:, :512] + w2[:, 512:1024] + w2[:, 1024:]
    z = lax.dot_general(t, wsum, (((1,), (1,)), ((), ())),
                        preferred_element_type=_f32)
    rowid = i * R + lax.broadcasted_iota(_i32, (R, 1), 0)
    zs = jnp.where(rowid < N, z * dv, 0.0)
    zs0_ref[...] = zs[:, :FH]
    zs1_ref[...] = zs[:, FH:]


def _mid(p0, p1, xs0, xs1, dinv2, W1, W2, b1):
    return pl.pallas_call(
        _mid_body,
        grid=(GRID,),
        in_specs=[
            pl.BlockSpec((R, FH), lambda i: (i, 0)),
            pl.BlockSpec((R, FH), lambda i: (i, 0)),
            pl.BlockSpec((R, FH), lambda i: (i, 0)),
            pl.BlockSpec((R, FH), lambda i: (i, 0)),
            pl.BlockSpec((R, 1), lambda i: (i, 0)),
            pl.BlockSpec((512, 256), lambda i: (0, 0)),
            pl.BlockSpec((256, 1536), lambda i: (0, 0)),
            pl.BlockSpec((1, 1, 512), lambda i: (0, 0, 0)),
        ],
        out_specs=[
            pl.BlockSpec((R, FH), lambda i: (i, 0)),
            pl.BlockSpec((R, FH), lambda i: (i, 0)),
        ],
        out_shape=[
            jax.ShapeDtypeStruct((NPAD, FH), _f32),
            jax.ShapeDtypeStruct((NPAD, FH), _f32),
        ],
    )(p0, p1, xs0, xs1, dinv2, W1, W2, b1)


def _final_body(q0_ref, q1_ref, zs0_ref, zs1_ref, dv_ref, b2_ref, out_ref):
    y2 = dv_ref[...] * jnp.concatenate(
        [q0_ref[...] + zs0_ref[...], q1_ref[...] + zs1_ref[...]], axis=1
    ) + b2_ref[0]
    r = jnp.maximum(y2, 0.0)
    out_ref[...] = jnp.concatenate([r, r, r], axis=1)


def _final(q0, q1, zs0, zs1, dinv2, b2):
    return pl.pallas_call(
        _final_body,
        grid=(GRID,),
        in_specs=[
            pl.BlockSpec((R, FH), lambda i: (i, 0)),
            pl.BlockSpec((R, FH), lambda i: (i, 0)),
            pl.BlockSpec((R, FH), lambda i: (i, 0)),
            pl.BlockSpec((R, FH), lambda i: (i, 0)),
            pl.BlockSpec((R, 1), lambda i: (i, 0)),
            pl.BlockSpec((1, 1, 256), lambda i: (0, 0, 0)),
        ],
        out_specs=pl.BlockSpec((R, 768), lambda i: (i, 0)),
        out_shape=jax.ShapeDtypeStruct((N, 768), _f32),
    )(q0, q1, zs0, zs1, dinv2, b2)


def kernel(x, edge_index, split, W1, W2, bias1, bias2):
    del split
    assert x.shape == (N, 256) and edge_index.shape == (2, E)
    edge4 = edge_index.astype(_i32).reshape(2, NS, NW, W)
    rowp3, colp3, dinv = _prep(edge4)
    col2 = colp3.reshape(NS * NW, W)
    dinv2 = dinv.reshape(NPAD, 1)
    xs0, xs1 = _scale(x, dinv2)
    p0, p1 = _propagate(xs0, xs1, rowp3, col2)
    zs0, zs1 = _mid(p0, p1, xs0, xs1, dinv2, W1, W2,
                    bias1.reshape(1, 1, 512))
    q0, q1 = _propagate(zs0, zs1, rowp3, col2)
    return _final(q0, q1, zs0, zs1, dinv2, bias2.reshape(1, 1, 256))
